# Initial kernel scaffold; baseline (speedup 1.0000x reference)
import jax
import jax.numpy as jnp
from jax.experimental import pallas as pl


def kernel(input, feature_params):
    raise NotImplementedError("write your pallas kernel here")


# speedup vs baseline: 185.8865x; 185.8865x over previous
import functools

import jax
import jax.numpy as jnp
from jax import lax
from jax.experimental import pallas as pl
from jax.experimental.pallas import tpu as pltpu
from jax.experimental.pallas import tpu_sc as plsc

L = 16
NW = 32
C = 4096
G = 128


@functools.lru_cache(maxsize=None)
def _build(n, res):
    per_w = n // NW
    n_chunks = per_w // C
    mesh = plsc.VectorSubcoreMesh(core_axis_name="c", subcore_axis_name="s")

    @functools.partial(
        pl.kernel,
        out_type=jax.ShapeDtypeStruct((n,), jnp.float32),
        mesh=mesh,
        scratch_types=[
            pltpu.VMEM((C,), jnp.float32),
            pltpu.VMEM((C,), jnp.int32),
            pltpu.VMEM((C,), jnp.int32),
            pltpu.VMEM((C,), jnp.float32),
            pltpu.VMEM((C,), jnp.float32),
            pltpu.VMEM((C,), jnp.float32),
            pltpu.VMEM((C,), jnp.float32),
            pltpu.SemaphoreType.DMA,
        ],
    )
    def grid_lookup(inp_hbm, tab_hbm, out_hbm, x_v, ilo_v, ihi_v, t_v,
                    lo_v, hi_v, o_v, sem):
        wid = lax.axis_index("s") * 2 + lax.axis_index("c")
        w_base = wid * per_w

        def chunk_body(g, _):
            base = w_base + g * C
            pltpu.sync_copy(inp_hbm.at[pl.ds(base, C)], x_v)

            def prep(i, _):
                x = x_v[pl.ds(i * L, L)]
                scaled = x * float(res - 1)
                low = jnp.clip(scaled.astype(jnp.int32), 0, res - 2)
                ilo_v[pl.ds(i * L, L)] = low
                ihi_v[pl.ds(i * L, L)] = low + 1
                t_v[pl.ds(i * L, L)] = scaled - low.astype(jnp.float32)
                return 0

            lax.fori_loop(0, C // L, prep, 0)

            def fire(j, _):
                o = j * G
                pltpu.async_copy(tab_hbm.at[ilo_v.at[pl.ds(o, G)]],
                                 lo_v.at[pl.ds(o, G)], sem)
                pltpu.async_copy(tab_hbm.at[ihi_v.at[pl.ds(o, G)]],
                                 hi_v.at[pl.ds(o, G)], sem)
                return 0

            lax.fori_loop(0, C // G, fire, 0)
            pltpu.make_async_copy(tab_hbm.at[pl.ds(0, C)], lo_v, sem).wait()
            pltpu.make_async_copy(tab_hbm.at[pl.ds(0, C)], hi_v, sem).wait()

            def lerp(i, _):
                s = pl.ds(i * L, L)
                t = t_v[s]
                o_v[s] = lo_v[s] * (1.0 - t) + hi_v[s] * t
                return 0

            lax.fori_loop(0, C // L, lerp, 0)
            pltpu.sync_copy(o_v, out_hbm.at[pl.ds(base, C)])
            return 0

        lax.fori_loop(0, n_chunks, chunk_body, 0)

    return grid_lookup


def kernel(input, feature_params):
    return _build(input.shape[0], feature_params.shape[0])(input, feature_params)


# speedup vs baseline: 215.8767x; 1.1613x over previous
import functools

import jax
import jax.numpy as jnp
from jax import lax
from jax.experimental import pallas as pl
from jax.experimental.pallas import tpu as pltpu
from jax.experimental.pallas import tpu_sc as plsc

L = 16
NW = 32
C = 4096
G = 128


@functools.lru_cache(maxsize=None)
def _build(n, res):
    per_w = n // NW
    n_chunks = per_w // C
    assert n_chunks >= 4 and n_chunks % 2 == 0
    mesh = plsc.VectorSubcoreMesh(core_axis_name="c", subcore_axis_name="s")

    buf = lambda dt: pltpu.VMEM((C,), dt)

    @functools.partial(
        pl.kernel,
        out_type=jax.ShapeDtypeStruct((n,), jnp.float32),
        mesh=mesh,
        scratch_types=[
            buf(jnp.float32), buf(jnp.int32), buf(jnp.float32),
            buf(jnp.float32), buf(jnp.float32),
            buf(jnp.float32), buf(jnp.int32), buf(jnp.float32),
            buf(jnp.float32), buf(jnp.float32),
            buf(jnp.float32),
            pltpu.SemaphoreType.DMA,
            pltpu.SemaphoreType.DMA,
            pltpu.SemaphoreType.DMA,
        ],
    )
    def grid_lookup(inp_hbm, tab_hbm, tab1_hbm, out_hbm,
                    xA, idxA, tA, loA, hiA, xB, idxB, tB, loB, hiB,
                    o_v, sem_in, semA, semB):
        wid = lax.axis_index("s") * 2 + lax.axis_index("c")
        w_base = wid * per_w

        A = (xA, idxA, tA, loA, hiA, semA)
        B = (xB, idxB, tB, loB, hiB, semB)

        def in_start(g, bufs):
            pltpu.async_copy(inp_hbm.at[pl.ds(w_base + g * C, C)],
                             bufs[0], sem_in)

        def in_wait():
            pltpu.make_async_copy(inp_hbm.at[pl.ds(0, C)], xA, sem_in).wait()

        def prep(bufs):
            x_v, idx_v, t_v = bufs[0], bufs[1], bufs[2]

            def body(i, _):
                s = pl.ds(i * L, L)
                scaled = x_v[s] * float(res - 1)
                low = jnp.clip(scaled.astype(jnp.int32), 0, res - 2)
                idx_v[s] = low
                t_v[s] = scaled - low.astype(jnp.float32)
                return 0

            lax.fori_loop(0, C // L, body, 0, unroll=4)

        def fire(bufs):
            idx_v, lo_v, hi_v, sem = bufs[1], bufs[3], bufs[4], bufs[5]

            def body(j, _):
                s = pl.ds(j * G, G)
                pltpu.async_copy(tab_hbm.at[idx_v.at[s]], lo_v.at[s], sem)
                pltpu.async_copy(tab1_hbm.at[idx_v.at[s]], hi_v.at[s], sem)
                return 0

            lax.fori_loop(0, C // G, body, 0, unroll=2)

        def drain(bufs):
            lo_v, hi_v, sem = bufs[3], bufs[4], bufs[5]
            pltpu.make_async_copy(tab_hbm.at[pl.ds(0, C)], lo_v, sem).wait()
            pltpu.make_async_copy(tab_hbm.at[pl.ds(0, C)], hi_v, sem).wait()

        def lerp_out(g, bufs):
            t_v, lo_v, hi_v = bufs[2], bufs[3], bufs[4]

            def body(i, _):
                s = pl.ds(i * L, L)
                t = t_v[s]
                o_v[s] = lo_v[s] * (1.0 - t) + hi_v[s] * t
                return 0

            lax.fori_loop(0, C // L, body, 0, unroll=4)
            pltpu.sync_copy(o_v, out_hbm.at[pl.ds(w_base + g * C, C)])

        last = n_chunks - 1

        in_start(0, A)
        in_wait()
        prep(A)
        fire(A)
        in_start(1, B)
        in_wait()
        prep(B)
        fire(B)
        in_start(2, A)
        drain(A)
        lerp_out(0, A)

        def pair_body(gg, _):
            g = 2 * gg + 2
            in_wait()
            prep(A)
            fire(A)
            in_start(jnp.minimum(g + 1, last), B)
            drain(B)
            lerp_out(g - 1, B)

            g2 = g + 1
            in_wait()
            prep(B)
            fire(B)
            in_start(jnp.minimum(g2 + 1, last), A)
            drain(A)
            lerp_out(g2 - 1, A)
            return 0

        lax.fori_loop(0, (n_chunks - 2) // 2, pair_body, 0)

        drain(B)
        lerp_out(last, B)
        in_wait()

    return grid_lookup


def kernel(input, feature_params):
    return _build(input.shape[0], feature_params.shape[0])(
        input, feature_params, feature_params[1:])


# speedup vs baseline: 231.7627x; 1.0736x over previous
import functools

import jax
import jax.numpy as jnp
from jax import lax
from jax.experimental import pallas as pl
from jax.experimental.pallas import tpu as pltpu
from jax.experimental.pallas import tpu_sc as plsc

L = 16
NW = 32
C = 4096
G = 128


@functools.lru_cache(maxsize=None)
def _build(n, res):
    per_w = n // NW
    n_chunks = per_w // C
    assert n_chunks >= 4 and n_chunks % 2 == 0
    mesh = plsc.VectorSubcoreMesh(core_axis_name="c", subcore_axis_name="s")

    buf = lambda dt: pltpu.VMEM((C,), dt)

    @functools.partial(
        pl.kernel,
        out_type=jax.ShapeDtypeStruct((n,), jnp.float32),
        mesh=mesh,
        scratch_types=[
            buf(jnp.float32), buf(jnp.int32), buf(jnp.float32),
            buf(jnp.float32), buf(jnp.float32),
            buf(jnp.float32), buf(jnp.int32), buf(jnp.float32),
            buf(jnp.float32), buf(jnp.float32),
            buf(jnp.float32),
            pltpu.SemaphoreType.DMA,
            pltpu.SemaphoreType.DMA,
            pltpu.SemaphoreType.DMA,
        ],
    )
    def grid_lookup(inp_hbm, tab_hbm, tab1_hbm, out_hbm,
                    xA, idxA, tA, loA, hiA, xB, idxB, tB, loB, hiB,
                    o_v, sem_in, semA, semB):
        wid = lax.axis_index("s") * 2 + lax.axis_index("c")
        w_base = wid * per_w

        A = (xA, idxA, tA, loA, hiA, semA)
        B = (xB, idxB, tB, loB, hiB, semB)

        def in_start(g, bufs):
            pltpu.async_copy(inp_hbm.at[pl.ds(w_base + g * C, C)],
                             bufs[0], sem_in)

        def in_wait():
            pltpu.make_async_copy(inp_hbm.at[pl.ds(0, C)], xA, sem_in).wait()

        def prep(bufs):
            x_v, idx_v, t_v = bufs[0], bufs[1], bufs[2]

            def body(i, _):
                s = pl.ds(i * L, L)
                scaled = x_v[s] * float(res - 1)
                low = jnp.clip(scaled.astype(jnp.int32), 0, res - 2)
                idx_v[s] = low
                t_v[s] = scaled - low.astype(jnp.float32)
                return 0

            lax.fori_loop(0, C // L, body, 0, unroll=4)

        def fire(bufs):
            idx_v, lo_v, hi_v, sem = bufs[1], bufs[3], bufs[4], bufs[5]
            pltpu.async_copy(tab_hbm.at[idx_v], lo_v, sem)
            pltpu.async_copy(tab1_hbm.at[idx_v], hi_v, sem)

        def drain(bufs):
            lo_v, hi_v, sem = bufs[3], bufs[4], bufs[5]
            pltpu.make_async_copy(tab_hbm.at[pl.ds(0, C)], lo_v, sem).wait()
            pltpu.make_async_copy(tab_hbm.at[pl.ds(0, C)], hi_v, sem).wait()

        def lerp_out(g, bufs):
            t_v, lo_v, hi_v = bufs[2], bufs[3], bufs[4]

            def body(i, _):
                s = pl.ds(i * L, L)
                t = t_v[s]
                o_v[s] = lo_v[s] * (1.0 - t) + hi_v[s] * t
                return 0

            lax.fori_loop(0, C // L, body, 0, unroll=4)
            pltpu.sync_copy(o_v, out_hbm.at[pl.ds(w_base + g * C, C)])

        last = n_chunks - 1

        in_start(0, A)
        in_wait()
        prep(A)
        fire(A)
        in_start(1, B)
        in_wait()
        prep(B)
        fire(B)
        in_start(2, A)
        drain(A)
        lerp_out(0, A)

        def pair_body(gg, _):
            g = 2 * gg + 2
            in_wait()
            prep(A)
            fire(A)
            in_start(jnp.minimum(g + 1, last), B)
            drain(B)
            lerp_out(g - 1, B)

            g2 = g + 1
            in_wait()
            prep(B)
            fire(B)
            in_start(jnp.minimum(g2 + 1, last), A)
            drain(A)
            lerp_out(g2 - 1, A)
            return 0

        lax.fori_loop(0, (n_chunks - 2) // 2, pair_body, 0)

        drain(B)
        lerp_out(last, B)
        in_wait()

    return grid_lookup


def kernel(input, feature_params):
    return _build(input.shape[0], feature_params.shape[0])(
        input, feature_params, feature_params[1:])
